# Initial kernel scaffold; baseline (speedup 1.0000x reference)
import jax
import jax.numpy as jnp
from jax.experimental import pallas as pl


def kernel(x, edge_index, edge_attr, Wl1, bl1, Wr1, br1, We1, att1, b1, Wl2, bl2, Wr2, br2, We2, att2, b2, Wf, bf):
    raise NotImplementedError("write your pallas kernel here")


# speedup vs baseline: 3.9756x; 3.9756x over previous
import functools

import jax
import jax.numpy as jnp
from jax import lax
from jax.experimental import pallas as pl
from jax.experimental.pallas import tpu as pltpu


def _mm_body(x_ref, w_ref, xt_ref, wt_ref, b_ref, o_ref, *, nk):
    k = pl.program_id(1)

    @pl.when(k == 0)
    def _():
        o_ref[...] = jnp.dot(xt_ref[...], wt_ref[...],
                             preferred_element_type=jnp.float32)

    o_ref[...] += jnp.dot(x_ref[...], w_ref[...],
                          preferred_element_type=jnp.float32)

    @pl.when(k == nk - 1)
    def _():
        o_ref[...] += b_ref[...]


def _matmul(x, w, b, bn=1024, bk=512):
    m, kdim = x.shape
    n = w.shape[1]
    bn = min(bn, n)
    bk = min(bk, (kdim // 128) * 128)
    kmain = (kdim // bk) * bk
    nk = kmain // bk
    ktail = kdim - kmain
    if ktail > 0:
        xt = jnp.pad(x[:, kmain:], ((0, 0), (0, 128 - ktail)))
        wt = jnp.pad(w[kmain:, :], ((0, 128 - ktail), (0, 0)))
    else:
        xt = jnp.zeros((m, 128), jnp.float32)
        wt = jnp.zeros((128, n), jnp.float32)
    nn = n // bn
    return pl.pallas_call(
        functools.partial(_mm_body, nk=nk),
        grid=(nn, nk),
        in_specs=[
            pl.BlockSpec((m, bk), lambda j, k: (0, k)),
            pl.BlockSpec((bk, bn), lambda j, k: (k, j)),
            pl.BlockSpec((m, 128), lambda j, k: (0, 0)),
            pl.BlockSpec((128, bn), lambda j, k: (0, j)),
            pl.BlockSpec((1, bn), lambda j, k: (0, j)),
        ],
        out_specs=pl.BlockSpec((m, bn), lambda j, k: (0, j)),
        out_shape=jax.ShapeDtypeStruct((m, n), jnp.float32),
    )(x, w, xt, wt, b.reshape(1, n))


def _alpha_body(src_ref, dst_ref, xl_ref, xr_ref, ea_ref, we_ref, att_ref,
                alpha_ref, gmax_ref, gl, *, B, HPG, C):
    i = pl.program_id(0)
    base = i * B

    def gather(k, _):
        s = src_ref[base + k]
        d = dst_ref[base + k]
        gl[k, :] = xl_ref[s, :] + xr_ref[d, :]
        return 0

    lax.fori_loop(0, B, gather, 0, unroll=8)

    ep = jnp.dot(ea_ref[...], we_ref[...], preferred_element_type=jnp.float32)
    u = gl[...] + ep
    u = jnp.where(u >= 0.0, u, 0.2 * u)
    p = u * att_ref[...]
    cols = [jnp.sum(p[:, h * C:(h + 1) * C], axis=1, keepdims=True)
            for h in range(HPG)]
    cols.append(jnp.zeros((B, 128 - HPG), jnp.float32))
    r = jnp.concatenate(cols, axis=1)
    alpha_ref[...] = r

    m = jnp.max(r, axis=0, keepdims=True)

    @pl.when(i == 0)
    def _():
        gmax_ref[...] = jnp.full((8, 128), -1e30, jnp.float32)

    gmax_ref[...] = jnp.maximum(gmax_ref[...], jnp.broadcast_to(m, (8, 128)))


def _edge_alpha(src, dst, xl, xr, ea_pad, we_pad, att_flat,
                *, B, HPG, C, nblk):
    n = xl.shape[0]
    colw = HPG * C
    grid_spec = pltpu.PrefetchScalarGridSpec(
        num_scalar_prefetch=2,
        grid=(nblk,),
        in_specs=[
            pl.BlockSpec((n, colw), lambda i, *_: (0, 0)),
            pl.BlockSpec((n, colw), lambda i, *_: (0, 0)),
            pl.BlockSpec((B, 128), lambda i, *_: (i, 0)),
            pl.BlockSpec((128, colw), lambda i, *_: (0, 0)),
            pl.BlockSpec((1, colw), lambda i, *_: (0, 0)),
        ],
        out_specs=[
            pl.BlockSpec((B, 128), lambda i, *_: (i, 0)),
            pl.BlockSpec((8, 128), lambda i, *_: (0, 0)),
        ],
        scratch_shapes=[
            pltpu.VMEM((B, colw), jnp.float32),
        ],
    )
    return pl.pallas_call(
        functools.partial(_alpha_body, B=B, HPG=HPG, C=C),
        grid_spec=grid_spec,
        out_shape=[
            jax.ShapeDtypeStruct((nblk * B, 128), jnp.float32),
            jax.ShapeDtypeStruct((8, 128), jnp.float32),
        ],
    )(src, dst, xl, xr, ea_pad, we_pad, att_flat)


def _agg_body(src_ref, dst_ref, alpha_ref, gmax_ref, dstv_ref, xl_ref, b_ref,
              out_ref, den_ref, wx, *, B, HPG, C, M, nblk, etot):
    i = pl.program_id(0)
    base = i * B

    @pl.when(i == 0)
    def _():
        out_ref[...] = jnp.zeros_like(out_ref)
        den_ref[...] = jnp.zeros_like(den_ref)

    w128 = jnp.exp(alpha_ref[...] - gmax_ref[0:1, :])
    eidx = base + lax.broadcasted_iota(jnp.int32, (B, 128), 0)
    w128 = jnp.where(eidx < etot, w128, 0.0)

    dcol = dstv_ref[:, 0:1]
    iota = lax.broadcasted_iota(jnp.int32, (B, M), 1)
    oh = jnp.where(dcol == iota, 1.0, 0.0)
    den_ref[...] += lax.dot_general(oh, w128, (((0,), (0,)), ((), ())),
                                    preferred_element_type=jnp.float32)

    cols = [jnp.broadcast_to(w128[:, h:h + 1], (B, C)) for h in range(HPG)]
    wx[...] = jnp.concatenate(cols, axis=1) if HPG > 1 else cols[0]

    def scatter(k, _):
        s = src_ref[base + k]
        d = dst_ref[base + k]
        out_ref[d, :] += xl_ref[s, :] * wx[k, :]
        return 0

    lax.fori_loop(0, B, scatter, 0, unroll=8)

    @pl.when(i == nblk - 1)
    def _():
        for h in range(HPG):
            dh = den_ref[0:out_ref.shape[0], h:h + 1]
            out_ref[:, h * C:(h + 1) * C] = (
                out_ref[:, h * C:(h + 1) * C] / dh + b_ref[0:1, h * C:(h + 1) * C])


def _edge_aggregate(src, dst, alpha, gmax, dstv, xl, bias,
                    *, B, HPG, C, nblk, etot):
    n = xl.shape[0]
    colw = HPG * C
    m = 2048
    grid_spec = pltpu.PrefetchScalarGridSpec(
        num_scalar_prefetch=2,
        grid=(nblk,),
        in_specs=[
            pl.BlockSpec((B, 128), lambda i, *_: (i, 0)),
            pl.BlockSpec((8, 128), lambda i, *_: (0, 0)),
            pl.BlockSpec((B, 128), lambda i, *_: (i, 0)),
            pl.BlockSpec((n, colw), lambda i, *_: (0, 0)),
            pl.BlockSpec((1, colw), lambda i, *_: (0, 0)),
        ],
        out_specs=[
            pl.BlockSpec((n, colw), lambda i, *_: (0, 0)),
            pl.BlockSpec((m, 128), lambda i, *_: (0, 0)),
        ],
        scratch_shapes=[
            pltpu.VMEM((B, colw), jnp.float32),
        ],
    )
    out, _ = pl.pallas_call(
        functools.partial(_agg_body, B=B, HPG=HPG, C=C, M=m, nblk=nblk,
                          etot=etot),
        grid_spec=grid_spec,
        out_shape=[
            jax.ShapeDtypeStruct((n, colw), jnp.float32),
            jax.ShapeDtypeStruct((m, 128), jnp.float32),
        ],
    )(src, dst, alpha, gmax, dstv, xl, bias.reshape(1, colw))
    return out


_EB = 512

def _gat_layer(xl, xr, src, dst, dstv, ea_pad, we, att, bias,
               *, B, H, C, nblk, etot):
    hg = 2 if H > 1 else 1
    hpg = H // hg
    colw = hpg * C
    we_pad = jnp.pad(we, ((0, 128 - we.shape[0]), (0, 0)))
    outs = []
    for g in range(hg):
        sl = slice(g * colw, (g + 1) * colw)
        xl_g = xl[:, sl]
        xr_g = xr[:, sl]
        att_g = att[g * hpg:(g + 1) * hpg].reshape(1, colw)
        alpha, gmax = _edge_alpha(src, dst, xl_g, xr_g, ea_pad,
                                  we_pad[:, sl], att_g,
                                  B=B, HPG=hpg, C=C, nblk=nblk)
        outs.append(_edge_aggregate(src, dst, alpha, gmax, dstv, xl_g,
                                    bias[sl], B=B, HPG=hpg, C=C,
                                    nblk=nblk, etot=etot))
    if hg == 1:
        return outs[0]
    return jnp.concatenate(outs, axis=1)


def kernel(x, edge_index, edge_attr, Wl1, bl1, Wr1, br1, We1, att1, b1,
           Wl2, bl2, Wr2, br2, We2, att2, b2, Wf, bf):
    n = x.shape[0]
    e = edge_index.shape[1]
    etot = e + n
    B = _EB
    nblk = -(-etot // B)
    ep_n = nblk * B

    loops = jnp.arange(n, dtype=edge_index.dtype)
    pad = ep_n - etot
    zpad = jnp.zeros((pad,), edge_index.dtype)
    src = jnp.concatenate([edge_index[0], loops, zpad])
    dst = jnp.concatenate([edge_index[1], loops, zpad])
    mean_attr = jnp.mean(edge_attr, axis=0, keepdims=True)
    ea = jnp.concatenate([
        edge_attr,
        jnp.broadcast_to(mean_attr, (n, edge_attr.shape[1])),
        jnp.zeros((pad, edge_attr.shape[1]), jnp.float32),
    ], axis=0)
    ea_pad = jnp.pad(ea, ((0, 0), (0, 128 - ea.shape[1])))
    dstv = jnp.pad(dst[:, None], ((0, 0), (0, 127)))

    h1, c = att1.shape

    xl1 = _matmul(x, Wl1, bl1)
    xr1 = _matmul(x, Wr1, br1)
    h = _gat_layer(xl1, xr1, src, dst, dstv, ea_pad, We1, att1, b1,
                   B=B, H=h1, C=c, nblk=nblk, etot=etot)

    xl2 = _matmul(h, Wl2, bl2)
    xr2 = _matmul(h, Wr2, br2)
    h2 = _gat_layer(xl2, xr2, src, dst, dstv, ea_pad, We2, att2, b2,
                    B=B, H=1, C=c, nblk=nblk, etot=etot)

    nout = Wf.shape[1]
    wf_pad = jnp.pad(Wf, ((0, 0), (0, 128 - nout)))
    bf_pad = jnp.pad(bf, (0, 128 - nout))
    out = _matmul(h2, wf_pad, bf_pad, bn=128, bk=512)
    return out[:, :nout]


# speedup vs baseline: 4.0221x; 1.0117x over previous
import functools

import jax
import jax.numpy as jnp
from jax import lax
from jax.experimental import pallas as pl
from jax.experimental.pallas import tpu as pltpu


def _mm_body(x_ref, w_ref, xt_ref, wt_ref, b_ref, o_ref, *, nk):
    k = pl.program_id(1)

    @pl.when(k == 0)
    def _():
        o_ref[...] = jnp.dot(xt_ref[...], wt_ref[...],
                             preferred_element_type=jnp.float32)

    o_ref[...] += jnp.dot(x_ref[...], w_ref[...],
                          preferred_element_type=jnp.float32)

    @pl.when(k == nk - 1)
    def _():
        o_ref[...] += b_ref[...]


def _matmul(x, w, b, bn=1024, bk=512):
    m, kdim = x.shape
    n = w.shape[1]
    bn = min(bn, n)
    bk = min(bk, (kdim // 128) * 128)
    kmain = (kdim // bk) * bk
    nk = kmain // bk
    ktail = kdim - kmain
    if ktail > 0:
        xt = jnp.pad(x[:, kmain:], ((0, 0), (0, 128 - ktail)))
        wt = jnp.pad(w[kmain:, :], ((0, 128 - ktail), (0, 0)))
    else:
        xt = jnp.zeros((m, 128), jnp.float32)
        wt = jnp.zeros((128, n), jnp.float32)
    nn = n // bn
    return pl.pallas_call(
        functools.partial(_mm_body, nk=nk),
        grid=(nn, nk),
        in_specs=[
            pl.BlockSpec((m, bk), lambda j, k: (0, k)),
            pl.BlockSpec((bk, bn), lambda j, k: (k, j)),
            pl.BlockSpec((m, 128), lambda j, k: (0, 0)),
            pl.BlockSpec((128, bn), lambda j, k: (0, j)),
            pl.BlockSpec((1, bn), lambda j, k: (0, j)),
        ],
        out_specs=pl.BlockSpec((m, bn), lambda j, k: (0, j)),
        out_shape=jax.ShapeDtypeStruct((m, n), jnp.float32),
    )(x, w, xt, wt, b.reshape(1, n))


def _alpha_body(src_ref, dst_ref, xl_ref, xr_ref, ea_ref, we_ref, att_ref,
                alpha_ref, gmax_ref, gl, *, B, HPG, C):
    i = pl.program_id(0)
    base = i * B

    def gather(k, _):
        s = src_ref[base + k]
        d = dst_ref[base + k]
        gl[k, :] = xl_ref[s, :] + xr_ref[d, :]
        return 0

    lax.fori_loop(0, B, gather, 0, unroll=8)

    ep = jnp.dot(ea_ref[...], we_ref[...], preferred_element_type=jnp.float32)
    u = gl[...] + ep
    u = jnp.where(u >= 0.0, u, 0.2 * u)
    p = u * att_ref[...]
    cols = [jnp.sum(p[:, h * C:(h + 1) * C], axis=1, keepdims=True)
            for h in range(HPG)]
    cols.append(jnp.zeros((B, 128 - HPG), jnp.float32))
    r = jnp.concatenate(cols, axis=1)
    alpha_ref[...] = r

    m = jnp.max(r, axis=0, keepdims=True)

    @pl.when(i == 0)
    def _():
        gmax_ref[...] = jnp.full((8, 128), -1e30, jnp.float32)

    gmax_ref[...] = jnp.maximum(gmax_ref[...], jnp.broadcast_to(m, (8, 128)))


def _edge_alpha(src, dst, xl, xr, ea_pad, we_pad, att_flat,
                *, B, HPG, C, nblk):
    n = xl.shape[0]
    colw = HPG * C
    grid_spec = pltpu.PrefetchScalarGridSpec(
        num_scalar_prefetch=2,
        grid=(nblk,),
        in_specs=[
            pl.BlockSpec((n, colw), lambda i, *_: (0, 0)),
            pl.BlockSpec((n, colw), lambda i, *_: (0, 0)),
            pl.BlockSpec((B, 128), lambda i, *_: (i, 0)),
            pl.BlockSpec((128, colw), lambda i, *_: (0, 0)),
            pl.BlockSpec((1, colw), lambda i, *_: (0, 0)),
        ],
        out_specs=[
            pl.BlockSpec((B, 128), lambda i, *_: (i, 0)),
            pl.BlockSpec((8, 128), lambda i, *_: (0, 0)),
        ],
        scratch_shapes=[
            pltpu.VMEM((B, colw), jnp.float32),
        ],
    )
    return pl.pallas_call(
        functools.partial(_alpha_body, B=B, HPG=HPG, C=C),
        grid_spec=grid_spec,
        out_shape=[
            jax.ShapeDtypeStruct((nblk * B, 128), jnp.float32),
            jax.ShapeDtypeStruct((8, 128), jnp.float32),
        ],
    )(src, dst, xl, xr, ea_pad, we_pad, att_flat)


def _agg_body(src_ref, dst_ref, alpha_ref, gmax_ref, dstv_ref, xl_ref, b_ref,
              out_ref, den_ref, wx, *, B, HPG, C, M, nblk, etot):
    i = pl.program_id(0)
    base = i * B

    @pl.when(i == 0)
    def _():
        out_ref[...] = jnp.zeros_like(out_ref)
        den_ref[...] = jnp.zeros_like(den_ref)

    w128 = jnp.exp(alpha_ref[...] - gmax_ref[0:1, :])
    eidx = base + lax.broadcasted_iota(jnp.int32, (B, 128), 0)
    w128 = jnp.where(eidx < etot, w128, 0.0)

    dcol = dstv_ref[:, 0:1]
    iota = lax.broadcasted_iota(jnp.int32, (B, M), 1)
    oh = jnp.where(dcol == iota, 1.0, 0.0)
    den_ref[...] += lax.dot_general(oh, w128, (((0,), (0,)), ((), ())),
                                    preferred_element_type=jnp.float32)

    cols = [jnp.broadcast_to(w128[:, h:h + 1], (B, C)) for h in range(HPG)]
    wx[...] = jnp.concatenate(cols, axis=1) if HPG > 1 else cols[0]

    def scatter(k, _):
        s = src_ref[base + k]
        d = dst_ref[base + k]
        out_ref[d, :] += xl_ref[s, :] * wx[k, :]
        return 0

    lax.fori_loop(0, B, scatter, 0, unroll=8)

    @pl.when(i == nblk - 1)
    def _():
        for h in range(HPG):
            dh = den_ref[0:out_ref.shape[0], h:h + 1]
            out_ref[:, h * C:(h + 1) * C] = (
                out_ref[:, h * C:(h + 1) * C] / dh + b_ref[0:1, h * C:(h + 1) * C])


def _edge_aggregate(src, dst, alpha, gmax, dstv, xl, bias,
                    *, B, HPG, C, nblk, etot):
    n = xl.shape[0]
    colw = HPG * C
    m = 2048
    grid_spec = pltpu.PrefetchScalarGridSpec(
        num_scalar_prefetch=2,
        grid=(nblk,),
        in_specs=[
            pl.BlockSpec((B, 128), lambda i, *_: (i, 0)),
            pl.BlockSpec((8, 128), lambda i, *_: (0, 0)),
            pl.BlockSpec((B, 128), lambda i, *_: (i, 0)),
            pl.BlockSpec((n, colw), lambda i, *_: (0, 0)),
            pl.BlockSpec((1, colw), lambda i, *_: (0, 0)),
        ],
        out_specs=[
            pl.BlockSpec((n, colw), lambda i, *_: (0, 0)),
            pl.BlockSpec((m, 128), lambda i, *_: (0, 0)),
        ],
        scratch_shapes=[
            pltpu.VMEM((B, colw), jnp.float32),
        ],
    )
    out, _ = pl.pallas_call(
        functools.partial(_agg_body, B=B, HPG=HPG, C=C, M=m, nblk=nblk,
                          etot=etot),
        grid_spec=grid_spec,
        out_shape=[
            jax.ShapeDtypeStruct((n, colw), jnp.float32),
            jax.ShapeDtypeStruct((m, 128), jnp.float32),
        ],
    )(src, dst, alpha, gmax, dstv, xl, bias.reshape(1, colw))
    return out


_EB = 1024

def _gat_layer(xl, xr, src, dst, dstv, ea_pad, we, att, bias,
               *, B, H, C, nblk, etot):
    hg = 2 if H > 1 else 1
    hpg = H // hg
    colw = hpg * C
    we_pad = jnp.pad(we, ((0, 128 - we.shape[0]), (0, 0)))
    outs = []
    for g in range(hg):
        sl = slice(g * colw, (g + 1) * colw)
        xl_g = xl[:, sl]
        xr_g = xr[:, sl]
        att_g = att[g * hpg:(g + 1) * hpg].reshape(1, colw)
        alpha, gmax = _edge_alpha(src, dst, xl_g, xr_g, ea_pad,
                                  we_pad[:, sl], att_g,
                                  B=B, HPG=hpg, C=C, nblk=nblk)
        outs.append(_edge_aggregate(src, dst, alpha, gmax, dstv, xl_g,
                                    bias[sl], B=B, HPG=hpg, C=C,
                                    nblk=nblk, etot=etot))
    if hg == 1:
        return outs[0]
    return jnp.concatenate(outs, axis=1)


def kernel(x, edge_index, edge_attr, Wl1, bl1, Wr1, br1, We1, att1, b1,
           Wl2, bl2, Wr2, br2, We2, att2, b2, Wf, bf):
    n = x.shape[0]
    e = edge_index.shape[1]
    etot = e + n
    B = _EB
    nblk = -(-etot // B)
    ep_n = nblk * B

    loops = jnp.arange(n, dtype=edge_index.dtype)
    pad = ep_n - etot
    zpad = jnp.zeros((pad,), edge_index.dtype)
    src = jnp.concatenate([edge_index[0], loops, zpad])
    dst = jnp.concatenate([edge_index[1], loops, zpad])
    mean_attr = jnp.mean(edge_attr, axis=0, keepdims=True)
    ea = jnp.concatenate([
        edge_attr,
        jnp.broadcast_to(mean_attr, (n, edge_attr.shape[1])),
        jnp.zeros((pad, edge_attr.shape[1]), jnp.float32),
    ], axis=0)
    ea_pad = jnp.pad(ea, ((0, 0), (0, 128 - ea.shape[1])))
    dstv = jnp.pad(dst[:, None], ((0, 0), (0, 127)))

    h1, c = att1.shape

    xl1 = _matmul(x, Wl1, bl1)
    xr1 = _matmul(x, Wr1, br1)
    h = _gat_layer(xl1, xr1, src, dst, dstv, ea_pad, We1, att1, b1,
                   B=B, H=h1, C=c, nblk=nblk, etot=etot)

    xl2 = _matmul(h, Wl2, bl2)
    xr2 = _matmul(h, Wr2, br2)
    h2 = _gat_layer(xl2, xr2, src, dst, dstv, ea_pad, We2, att2, b2,
                    B=B, H=1, C=c, nblk=nblk, etot=etot)

    nout = Wf.shape[1]
    wf_pad = jnp.pad(Wf, ((0, 0), (0, 128 - nout)))
    bf_pad = jnp.pad(bf, (0, 128 - nout))
    out = _matmul(h2, wf_pad, bf_pad, bn=128, bk=512)
    return out[:, :nout]


# speedup vs baseline: 4.0505x; 1.0071x over previous
import functools

import jax
import jax.numpy as jnp
from jax import lax
from jax.experimental import pallas as pl
from jax.experimental.pallas import tpu as pltpu


def _mm_body(x_ref, w_ref, xt_ref, wt_ref, b_ref, o_ref, *, nk):
    k = pl.program_id(1)

    @pl.when(k == 0)
    def _():
        o_ref[...] = jnp.dot(xt_ref[...], wt_ref[...],
                             preferred_element_type=jnp.float32)

    o_ref[...] += jnp.dot(x_ref[...], w_ref[...],
                          preferred_element_type=jnp.float32)

    @pl.when(k == nk - 1)
    def _():
        o_ref[...] += b_ref[...]


def _matmul(x, w, b, bn=1024, bk=512):
    m, kdim = x.shape
    n = w.shape[1]
    bn = min(bn, n)
    bk = min(bk, (kdim // 128) * 128)
    kmain = (kdim // bk) * bk
    nk = kmain // bk
    ktail = kdim - kmain
    if ktail > 0:
        xt = jnp.pad(x[:, kmain:], ((0, 0), (0, 128 - ktail)))
        wt = jnp.pad(w[kmain:, :], ((0, 128 - ktail), (0, 0)))
    else:
        xt = jnp.zeros((m, 128), jnp.float32)
        wt = jnp.zeros((128, n), jnp.float32)
    nn = n // bn
    return pl.pallas_call(
        functools.partial(_mm_body, nk=nk),
        grid=(nn, nk),
        in_specs=[
            pl.BlockSpec((m, bk), lambda j, k: (0, k)),
            pl.BlockSpec((bk, bn), lambda j, k: (k, j)),
            pl.BlockSpec((m, 128), lambda j, k: (0, 0)),
            pl.BlockSpec((128, bn), lambda j, k: (0, j)),
            pl.BlockSpec((1, bn), lambda j, k: (0, j)),
        ],
        out_specs=pl.BlockSpec((m, bn), lambda j, k: (0, j)),
        out_shape=jax.ShapeDtypeStruct((m, n), jnp.float32),
    )(x, w, xt, wt, b.reshape(1, n))


def _alpha_body(src_ref, dst_ref, xl_ref, xr_ref, ea_ref, we_ref, att_ref,
                alpha_ref, gmax_ref, gl, *, B, HPG, C):
    i = pl.program_id(0)
    base = i * B

    def gather(k, _):
        s = src_ref[base + k]
        d = dst_ref[base + k]
        gl[k, :] = xl_ref[s, :] + xr_ref[d, :]
        return 0

    lax.fori_loop(0, B, gather, 0, unroll=16)

    ep = jnp.dot(ea_ref[...], we_ref[...], preferred_element_type=jnp.float32)
    u = gl[...] + ep
    u = jnp.where(u >= 0.0, u, 0.2 * u)
    p = u * att_ref[...]
    cols = [jnp.sum(p[:, h * C:(h + 1) * C], axis=1, keepdims=True)
            for h in range(HPG)]
    cols.append(jnp.zeros((B, 128 - HPG), jnp.float32))
    r = jnp.concatenate(cols, axis=1)
    alpha_ref[...] = r

    m = jnp.max(r, axis=0, keepdims=True)

    @pl.when(i == 0)
    def _():
        gmax_ref[...] = jnp.full((8, 128), -1e30, jnp.float32)

    gmax_ref[...] = jnp.maximum(gmax_ref[...], jnp.broadcast_to(m, (8, 128)))


def _edge_alpha(src, dst, xl, xr, ea_pad, we_pad, att_flat,
                *, B, HPG, C, nblk):
    n = xl.shape[0]
    colw = HPG * C
    grid_spec = pltpu.PrefetchScalarGridSpec(
        num_scalar_prefetch=2,
        grid=(nblk,),
        in_specs=[
            pl.BlockSpec((n, colw), lambda i, *_: (0, 0)),
            pl.BlockSpec((n, colw), lambda i, *_: (0, 0)),
            pl.BlockSpec((B, 128), lambda i, *_: (i, 0)),
            pl.BlockSpec((128, colw), lambda i, *_: (0, 0)),
            pl.BlockSpec((1, colw), lambda i, *_: (0, 0)),
        ],
        out_specs=[
            pl.BlockSpec((B, 128), lambda i, *_: (i, 0)),
            pl.BlockSpec((8, 128), lambda i, *_: (0, 0)),
        ],
        scratch_shapes=[
            pltpu.VMEM((B, colw), jnp.float32),
        ],
    )
    return pl.pallas_call(
        functools.partial(_alpha_body, B=B, HPG=HPG, C=C),
        grid_spec=grid_spec,
        out_shape=[
            jax.ShapeDtypeStruct((nblk * B, 128), jnp.float32),
            jax.ShapeDtypeStruct((8, 128), jnp.float32),
        ],
    )(src, dst, xl, xr, ea_pad, we_pad, att_flat)


def _agg_body(src_ref, dst_ref, alpha_ref, gmax_ref, dstv_ref, xl_ref, b_ref,
              out_ref, den_ref, wx, *, B, HPG, C, M, nblk, etot):
    i = pl.program_id(0)
    base = i * B

    @pl.when(i == 0)
    def _():
        out_ref[...] = jnp.zeros_like(out_ref)
        den_ref[...] = jnp.zeros_like(den_ref)

    w128 = jnp.exp(alpha_ref[...] - gmax_ref[0:1, :])
    eidx = base + lax.broadcasted_iota(jnp.int32, (B, 128), 0)
    w128 = jnp.where(eidx < etot, w128, 0.0)

    dcol = dstv_ref[:, 0:1]
    iota = lax.broadcasted_iota(jnp.int32, (B, M), 1)
    oh = jnp.where(dcol == iota, 1.0, 0.0)
    den_ref[...] += lax.dot_general(oh, w128, (((0,), (0,)), ((), ())),
                                    preferred_element_type=jnp.float32)

    cols = [jnp.broadcast_to(w128[:, h:h + 1], (B, C)) for h in range(HPG)]
    wx[...] = jnp.concatenate(cols, axis=1) if HPG > 1 else cols[0]

    def scatter(k, _):
        s = src_ref[base + k]
        d = dst_ref[base + k]
        out_ref[d, :] += xl_ref[s, :] * wx[k, :]
        return 0

    lax.fori_loop(0, B, scatter, 0, unroll=16)

    @pl.when(i == nblk - 1)
    def _():
        for h in range(HPG):
            dh = den_ref[0:out_ref.shape[0], h:h + 1]
            out_ref[:, h * C:(h + 1) * C] = (
                out_ref[:, h * C:(h + 1) * C] / dh + b_ref[0:1, h * C:(h + 1) * C])


def _edge_aggregate(src, dst, alpha, gmax, dstv, xl, bias,
                    *, B, HPG, C, nblk, etot):
    n = xl.shape[0]
    colw = HPG * C
    m = 2048
    grid_spec = pltpu.PrefetchScalarGridSpec(
        num_scalar_prefetch=2,
        grid=(nblk,),
        in_specs=[
            pl.BlockSpec((B, 128), lambda i, *_: (i, 0)),
            pl.BlockSpec((8, 128), lambda i, *_: (0, 0)),
            pl.BlockSpec((B, 128), lambda i, *_: (i, 0)),
            pl.BlockSpec((n, colw), lambda i, *_: (0, 0)),
            pl.BlockSpec((1, colw), lambda i, *_: (0, 0)),
        ],
        out_specs=[
            pl.BlockSpec((n, colw), lambda i, *_: (0, 0)),
            pl.BlockSpec((m, 128), lambda i, *_: (0, 0)),
        ],
        scratch_shapes=[
            pltpu.VMEM((B, colw), jnp.float32),
        ],
    )
    out, _ = pl.pallas_call(
        functools.partial(_agg_body, B=B, HPG=HPG, C=C, M=m, nblk=nblk,
                          etot=etot),
        grid_spec=grid_spec,
        out_shape=[
            jax.ShapeDtypeStruct((n, colw), jnp.float32),
            jax.ShapeDtypeStruct((m, 128), jnp.float32),
        ],
    )(src, dst, alpha, gmax, dstv, xl, bias.reshape(1, colw))
    return out


_EB = 1024

def _gat_layer(xl, xr, src, dst, dstv, ea_pad, we, att, bias,
               *, B, H, C, nblk, etot):
    hg = 2 if H > 1 else 1
    hpg = H // hg
    colw = hpg * C
    we_pad = jnp.pad(we, ((0, 128 - we.shape[0]), (0, 0)))
    outs = []
    for g in range(hg):
        sl = slice(g * colw, (g + 1) * colw)
        xl_g = xl[:, sl]
        xr_g = xr[:, sl]
        att_g = att[g * hpg:(g + 1) * hpg].reshape(1, colw)
        alpha, gmax = _edge_alpha(src, dst, xl_g, xr_g, ea_pad,
                                  we_pad[:, sl], att_g,
                                  B=B, HPG=hpg, C=C, nblk=nblk)
        outs.append(_edge_aggregate(src, dst, alpha, gmax, dstv, xl_g,
                                    bias[sl], B=B, HPG=hpg, C=C,
                                    nblk=nblk, etot=etot))
    if hg == 1:
        return outs[0]
    return jnp.concatenate(outs, axis=1)


def kernel(x, edge_index, edge_attr, Wl1, bl1, Wr1, br1, We1, att1, b1,
           Wl2, bl2, Wr2, br2, We2, att2, b2, Wf, bf):
    n = x.shape[0]
    e = edge_index.shape[1]
    etot = e + n
    B = _EB
    nblk = -(-etot // B)
    ep_n = nblk * B

    loops = jnp.arange(n, dtype=edge_index.dtype)
    pad = ep_n - etot
    zpad = jnp.zeros((pad,), edge_index.dtype)
    src = jnp.concatenate([edge_index[0], loops, zpad])
    dst = jnp.concatenate([edge_index[1], loops, zpad])
    mean_attr = jnp.mean(edge_attr, axis=0, keepdims=True)
    ea = jnp.concatenate([
        edge_attr,
        jnp.broadcast_to(mean_attr, (n, edge_attr.shape[1])),
        jnp.zeros((pad, edge_attr.shape[1]), jnp.float32),
    ], axis=0)
    ea_pad = jnp.pad(ea, ((0, 0), (0, 128 - ea.shape[1])))
    dstv = jnp.pad(dst[:, None], ((0, 0), (0, 127)))

    h1, c = att1.shape

    xl1 = _matmul(x, Wl1, bl1)
    xr1 = _matmul(x, Wr1, br1)
    h = _gat_layer(xl1, xr1, src, dst, dstv, ea_pad, We1, att1, b1,
                   B=B, H=h1, C=c, nblk=nblk, etot=etot)

    xl2 = _matmul(h, Wl2, bl2)
    xr2 = _matmul(h, Wr2, br2)
    h2 = _gat_layer(xl2, xr2, src, dst, dstv, ea_pad, We2, att2, b2,
                    B=B, H=1, C=c, nblk=nblk, etot=etot)

    nout = Wf.shape[1]
    wf_pad = jnp.pad(Wf, ((0, 0), (0, 128 - nout)))
    bf_pad = jnp.pad(bf, (0, 128 - nout))
    out = _matmul(h2, wf_pad, bf_pad, bn=128, bk=512)
    return out[:, :nout]
